# baseline (speedup 1.0000x reference)
import jax
import jax.numpy as jnp
from jax import lax
from jax.experimental import pallas as pl
from jax.experimental.pallas import tpu as pltpu
from jax.experimental.pallas import tpu_sc as plsc

NUM_ROWS = 16384 * 50
DIM = 32
NC, NS = 2, 16
NW = NC * NS
RPW = NUM_ROWS // NW

G = 256
NCHUNK = RPW // G


def _emb_body(idx_hbm, table_hbm, out_hbm, idx_v,
              srow0, srow1, rows0, rows1, stage0, stage1,
              gsem0, gsem1, osem0, osem1):
    srow = (srow0, srow1)
    rows = (rows0, rows1)
    stage = (stage0, stage1)
    gsem = (gsem0, gsem1)
    osem = (osem0, osem1)

    wid = lax.axis_index("s") * NC + lax.axis_index("c")
    base = wid * RPW
    pltpu.sync_copy(idx_hbm.at[pl.ds(base, RPW)], idx_v)

    iota = lax.iota(jnp.int32, 16)
    st_iota = iota * DIM

    def compute_srow(s, p):
        for k in range(2):
            for i in range(8):
                v = idx_v[pl.ds(s * G + k * 128 + i * 16, 16)]
                srow[p][k, pl.ds(i * 16, 16)] = v >> 2

    def gather_descs(s, p):
        for k in range(2):
            yield pltpu.make_async_copy(
                table_hbm.at[srow[p].at[k]],
                rows[p].at[pl.ds(k * 128, 128)],
                gsem[p],
            )

    def out_desc(s, p):
        return pltpu.make_async_copy(
            stage[p],
            out_hbm.at[pl.ds((base + s * G) * DIM, G * DIM)],
            osem[p],
        )

    def extract(s, p):
        def grp(g, carry):
            qv = (idx_v[pl.ds(s * G + g * 16, 16)] & 3) * DIM
            rid = g * 16 + iota
            sbase = g * 16 * DIM + st_iota
            for j in range(DIM):
                val = plsc.load_gather(rows[p], [rid, qv + j])
                plsc.store_scatter(stage[p], [sbase + j], val)
            return carry
        lax.fori_loop(0, G // 16, grp, 0)

    def do_step(s, p, fire_next, wait_out):
        q = 1 - p
        if fire_next:
            compute_srow(s + 1, q)
            for d in gather_descs(s + 1, q):
                d.start()
        for d in gather_descs(s, p):
            d.wait()
        if wait_out:
            out_desc(s - 2, p).wait()
        extract(s, p)
        out_desc(s, p).start()

    compute_srow(0, 0)
    for d in gather_descs(0, 0):
        d.start()
    do_step(0, 0, True, False)
    do_step(1, 1, True, False)

    def body(j, carry):
        s = 2 + j * 2
        do_step(s, 0, True, True)
        do_step(s + 1, 1, True, True)
        return carry
    lax.fori_loop(0, (NCHUNK - 4) // 2, body, 0)

    do_step(NCHUNK - 2, 0, True, True)
    do_step(NCHUNK - 1, 1, False, True)
    out_desc(NCHUNK - 2, 0).wait()
    out_desc(NCHUNK - 1, 1).wait()


_emb = pl.kernel(
    _emb_body,
    out_type=jax.ShapeDtypeStruct((NUM_ROWS * DIM,), jnp.float32),
    mesh=plsc.VectorSubcoreMesh(
        core_axis_name="c", subcore_axis_name="s", num_cores=NC, num_subcores=NS
    ),
    scratch_types=[
        pltpu.VMEM((RPW,), jnp.int32),
        pltpu.VMEM((2, 128), jnp.int32),
        pltpu.VMEM((2, 128), jnp.int32),
        pltpu.VMEM((G, 128), jnp.float32),
        pltpu.VMEM((G, 128), jnp.float32),
        pltpu.VMEM((G * DIM,), jnp.float32),
        pltpu.VMEM((G * DIM,), jnp.float32),
        pltpu.SemaphoreType.DMA,
        pltpu.SemaphoreType.DMA,
        pltpu.SemaphoreType.DMA,
        pltpu.SemaphoreType.DMA,
    ],
    compiler_params=pltpu.CompilerParams(
        use_tc_tiling_on_sc=True, needs_layout_passes=False
    ),
)


def kernel(input, weight):
    idx = input.reshape(-1).astype(jnp.int32)
    table4 = weight.reshape(250000, 128)
    out = _emb(idx, table4)
    return out.reshape(input.shape + (DIM,))


# speedup vs baseline: 1.1449x; 1.1449x over previous
import jax
import jax.numpy as jnp
from jax import lax
from jax.experimental import pallas as pl
from jax.experimental.pallas import tpu as pltpu
from jax.experimental.pallas import tpu_sc as plsc

NUM_ROWS = 16384 * 50
GROUP = 128
NC, NS = 2, 16
NW = NC * NS
RPW = NUM_ROWS // NW
GPW = RPW // GROUP
DIM = 32

K = 5
SUP = K * GROUP
NSUP = GPW // K
NB = 4


def _emb_body(idx_hbm, table_hbm, out_hbm, idx_v, rows_v, gsems, osems):
    wid = lax.axis_index("s") * NC + lax.axis_index("c")
    base = wid * RPW
    pltpu.sync_copy(idx_hbm.at[pl.ds(base, RPW)], idx_v)

    def gather_descs(t, p):
        for k in range(K):
            yield pltpu.make_async_copy(
                table_hbm.at[idx_v.at[pl.ds((t * K + k) * GROUP, GROUP)]],
                rows_v.at[p].at[pl.ds(k * GROUP, GROUP)],
                gsems.at[p],
            )

    def out_desc(t, p):
        return pltpu.make_async_copy(
            rows_v.at[p],
            out_hbm.at[pl.ds(base + t * SUP, SUP)],
            osems.at[p],
        )

    def do_step(s, p, fire_next, wait_out):
        if fire_next:
            q = (p + 1) % NB
            if wait_out:
                out_desc(s + 1 - NB, q).wait()
            for d in gather_descs(s + 1, q):
                d.start()
        for d in gather_descs(s, p):
            d.wait()
        out_desc(s, p).start()

    for d in gather_descs(0, 0):
        d.start()
    for s in range(NB):
        do_step(s, s % NB, fire_next=True, wait_out=(s == NB - 1))
    def body(j, carry):
        i = NB + j * NB
        for p in range(NB):
            do_step(i + p, p, fire_next=True, wait_out=True)
        return carry
    lax.fori_loop(0, (NSUP - 2 * NB) // NB, body, 0)
    for s in range(NSUP - NB, NSUP):
        do_step(s, s % NB, fire_next=(s + 1 < NSUP), wait_out=True)
    for s in range(NSUP - NB, NSUP):
        out_desc(s, s % NB).wait()


_emb = pl.kernel(
    _emb_body,
    out_type=jax.ShapeDtypeStruct((NUM_ROWS, DIM), jnp.float32),
    mesh=plsc.VectorSubcoreMesh(
        core_axis_name="c", subcore_axis_name="s", num_cores=NC, num_subcores=NS
    ),
    scratch_types=[
        pltpu.VMEM((RPW,), jnp.int32),
        pltpu.VMEM((NB, SUP, DIM), jnp.float32),
        pltpu.SemaphoreType.DMA((NB,)),
        pltpu.SemaphoreType.DMA((NB,)),
    ],
    compiler_params=pltpu.CompilerParams(use_tc_tiling_on_sc=False),
)


def kernel(input, weight):
    idx = input.reshape(-1).astype(jnp.int32)
    out = _emb(idx, weight)
    return out.reshape(input.shape + (DIM,))
